# baseline (speedup 1.0000x reference)
import functools

import jax
import jax.numpy as jnp
from jax import lax
from jax.experimental import pallas as pl
from jax.experimental.pallas import tpu as pltpu
from jax.experimental.pallas import tpu_sc as plsc

_N = 10000
_E = 320000
_F = 128
_H = 512
_G = 64

_NC = 2
_NS = 16
_NW = _NC * _NS
_K = 128
_CPS = 40
_C0 = 40
_C1 = 120
_CH_MAX = max(_C0, _C1)
_E0 = _NS * _C0 * _K
_E1 = _NS * _C1 * _K
_EPAD = _E0 + _E1
_NPAD = 10240
_RPT = _NPAD // _NS

_BM = 1000


@functools.cache
def _get_mp_pass(nt):
    return pl.kernel(
        functools.partial(_mp_body, nt),
        out_type=jax.ShapeDtypeStruct((nt, _NC, _NPAD, _F), jnp.float32),
        mesh=plsc.VectorSubcoreMesh(core_axis_name="c", subcore_axis_name="s",
                                    num_cores=_NC, num_subcores=_NS),
        scratch_types=[
            pltpu.VMEM_SHARED((_NPAD, _F), jnp.float32),
            pltpu.VMEM((_CPS, _K), jnp.int32),
            pltpu.VMEM((_CPS, _K), jnp.int32),
            pltpu.VMEM((_CPS, _K), jnp.float32),
            pltpu.VMEM((2, _K, _F), jnp.float32),
            pltpu.SemaphoreType.DMA,
            pltpu.SemaphoreType.DMA,
            pltpu.SemaphoreType.DMA,
            pltpu.SemaphoreType.DMA,
        ],
    )


def _mp_body(nt, tables, src, dst, w, out, accum, src_t, dst_t, w_t,
             rows_v, sg0, sg1, ss0, ss1):
    cid = lax.axis_index("c")
    sid = lax.axis_index("s")
    wid = cid * _NS + sid
    nstg = jnp.where(cid == 0, _C0 // _CPS, _C1 // _CPS)

    sgs = (sg0, sg1)
    sss = (ss0, ss1)

    @pl.loop(0, nt)
    def run_pass(t):
        @pl.loop(0, _K)
        def _zr(r):
            for v in range(_F // 16):
                rows_v[0, r, pl.ds(v * 16, 16)] = jnp.zeros((16,), jnp.float32)

        for j in range(_RPT // _K):
            pltpu.sync_copy(rows_v.at[0],
                            accum.at[pl.ds(sid * _RPT + j * _K, _K)])
        plsc.subcore_barrier()

        def gather(ci, b):
            return pltpu.async_copy(tables.at[t].at[src_t.at[ci]],
                                    rows_v.at[b], sgs[b])

        def gather_wait(ci, b):
            pltpu.make_async_copy(tables.at[t].at[src_t.at[ci]],
                                  rows_v.at[b], sgs[b]).wait()

        def scatter(ci, b):
            return pltpu.async_copy(rows_v.at[b], accum.at[dst_t.at[ci]],
                                    sss[b], add=True)

        def scatter_wait(ci, b):
            pltpu.make_async_copy(rows_v.at[b], accum.at[dst_t.at[ci]],
                                  sss[b]).wait()

        @pl.loop(0, nstg)
        def _stage(s):
            stg = pl.ds(s * _CPS, _CPS)
            pltpu.sync_copy(src.at[wid, stg], src_t)
            pltpu.sync_copy(dst.at[wid, stg], dst_t)
            pltpu.sync_copy(w.at[wid, stg], w_t)

            gather(0, 0)

            @pl.loop(0, _CPS, step=2)
            def _it(i):
                for b in range(2):
                    ci = i + b
                    gather_wait(ci, b)

                    @plsc.parallel_loop(0, _K // 16)
                    def _grp(g):
                        wvec = w_t[ci, pl.ds(g * 16, 16)]
                        for j in range(16):
                            wk = wvec[j]
                            k = g * 16 + j
                            for v in range(_F // 16):
                                sl = pl.ds(v * 16, 16)
                                rows_v[b, k, sl] = rows_v[b, k, sl] * wk

                    scatter(ci, b)

                    @pl.when(ci < _CPS - 1)
                    def _prefetch():
                        @pl.when(ci >= 1)
                        def _drain():
                            scatter_wait(ci - 1, b ^ 1)
                        gather(ci + 1, b ^ 1)

            scatter_wait(_CPS - 2, 0)
            scatter_wait(_CPS - 1, 1)

        plsc.subcore_barrier()
        pltpu.sync_copy(accum.at[pl.ds(sid * _RPT, _RPT)],
                        out.at[t, cid, pl.ds(sid * _RPT, _RPT)])


def _dense1_body(p_ref, x_ref, wrel_ref, b_ref, wroot_ref, out_ref):
    s = p_ref[0, 0] + p_ref[0, 1]
    xb = x_ref[...]
    for c in range(4):
        wslice = slice(c * _F, (c + 1) * _F)
        acc = jnp.dot(s, wrel_ref[:, wslice], preferred_element_type=jnp.float32)
        acc = acc + jnp.dot(xb, wroot_ref[:, wslice],
                            preferred_element_type=jnp.float32)
        acc = acc + b_ref[0, wslice][None, :]
        out_ref[c] = jnp.maximum(acc, 0.0)


def _dense1(p, x, w_rel, b, w_root):
    return pl.pallas_call(
        _dense1_body,
        grid=(_N // _BM,),
        in_specs=[
            pl.BlockSpec((1, _NC, _BM, _F), lambda m: (0, 0, m, 0)),
            pl.BlockSpec((_BM, _F), lambda m: (m, 0)),
            pl.BlockSpec((_F, _H), lambda m: (0, 0)),
            pl.BlockSpec((1, _H), lambda m: (0, 0)),
            pl.BlockSpec((_F, _H), lambda m: (0, 0)),
        ],
        out_specs=pl.BlockSpec((4, _BM, _F), lambda m: (0, m, 0)),
        out_shape=jax.ShapeDtypeStruct((4, _N, _F), jnp.float32),
    )(p, x, w_rel, b.reshape(1, _H), w_root)


def _dense2_body(q_ref, h_ref, wrel_ref, b_ref, wroot_ref,
                 bt_ref, wl1, bl1, wl2, bl2, wl3, bl3, out_ref,
                 pooled, counts):
    m = pl.program_id(0)
    nblocks = pl.num_programs(0)

    acc = jnp.broadcast_to(b_ref[0][None, :], (_BM, _H))
    for c in range(4):
        ksl = slice(c * _F, (c + 1) * _F)
        aggc = q_ref[c, 0] + q_ref[c, 1]
        acc = acc + jnp.dot(aggc, wrel_ref[ksl, :],
                            preferred_element_type=jnp.float32)
        acc = acc + jnp.dot(h_ref[c], wroot_ref[ksl, :],
                            preferred_element_type=jnp.float32)
    hout = jnp.maximum(acc, 0.0)

    bt = bt_ref[0, 0, :]
    onehot_t = (lax.broadcasted_iota(jnp.int32, (128, _BM), 0)
                == bt[None, :]).astype(jnp.float32)

    @pl.when(m == 0)
    def _init():
        pooled[...] = jnp.zeros_like(pooled)
        counts[...] = jnp.zeros_like(counts)

    pooled[...] += jnp.dot(onehot_t, hout, preferred_element_type=jnp.float32)
    counts[...] += jnp.broadcast_to(
        jnp.sum(onehot_t, axis=1, keepdims=True), (128, 128))

    @pl.when(m == nblocks - 1)
    def _final():
        cnt = counts[:, 0:1]
        mean = pooled[...] / jnp.maximum(cnt, 1.0)
        r = jnp.maximum(jnp.dot(mean, wl1[...],
                                preferred_element_type=jnp.float32)
                        + bl1[0][None, :], 0.0)
        r = jnp.maximum(jnp.dot(r, wl2[...],
                                preferred_element_type=jnp.float32)
                        + bl2[0][None, :], 0.0)
        o = jnp.dot(r, wl3[...], preferred_element_type=jnp.float32) \
            + bl3[0][None, :]
        out_ref[...] = jnp.broadcast_to(o[:_G, :], (_G, 128))


def _dense2(q, h1s, w_rel, b, w_root, bt3, wl1, bl1, wl2, bl2, wl3, bl3):
    full2 = lambda a, b_: pl.BlockSpec((a, b_), lambda m: (0, 0))
    return pl.pallas_call(
        _dense2_body,
        grid=(_N // _BM,),
        in_specs=[
            pl.BlockSpec((4, _NC, _BM, _F), lambda m: (0, 0, m, 0)),
            pl.BlockSpec((4, _BM, _F), lambda m: (0, m, 0)),
            full2(_H, _H),
            full2(1, _H),
            full2(_H, _H),
            pl.BlockSpec((1, 1, _BM), lambda m: (m, 0, 0)),
            full2(_H, 64),
            full2(1, 64),
            full2(64, 16),
            full2(1, 16),
            full2(16, 1),
            full2(1, 1),
        ],
        out_specs=pl.BlockSpec((_G, 128), lambda m: (0, 0)),
        out_shape=jax.ShapeDtypeStruct((_G, 128), jnp.float32),
        scratch_shapes=[
            pltpu.VMEM((128, _H), jnp.float32),
            pltpu.VMEM((128, 128), jnp.float32),
        ],
    )(q, h1s, w_rel, b.reshape(1, _H), w_root, bt3,
      wl1, bl1.reshape(1, 64), wl2, bl2.reshape(1, 16),
      wl3, bl3.reshape(1, 1))


def kernel(x, edge_index, edge_attr, batch, W1_rel, b1, W1_root, W2_rel, b2,
           W2_root, Wl1, bl1, Wl2, bl2, Wl3, bl3):
    def split3(a):
        a = jnp.pad(a, (0, _EPAD - _E))
        c0 = a[:_E0].reshape(_NS, _C0, _K)
        c0 = jnp.pad(c0, ((0, 0), (0, _CH_MAX - _C0), (0, 0)))
        c1 = a[_E0:].reshape(_NS, _C1, _K)
        c1 = jnp.pad(c1, ((0, 0), (0, _CH_MAX - _C1), (0, 0)))
        return jnp.concatenate([c0, c1], axis=0)

    src = split3(edge_index[0])
    dst = split3(edge_index[1])
    w = split3(edge_attr)

    p = _get_mp_pass(1)(x[None], src, dst, w)
    h1s = _dense1(p, x, W1_rel, b1, W1_root)
    q = _get_mp_pass(4)(h1s, src, dst, w)
    bt3 = batch.reshape(_N // _BM, 1, _BM)
    out = _dense2(q, h1s, W2_rel, b2, W2_root, bt3,
                  Wl1, bl1, Wl2, bl2, Wl3, bl3)
    return out[:, :1]


# speedup vs baseline: 1.2944x; 1.2944x over previous
import functools

import jax
import jax.numpy as jnp
from jax import lax
from jax.experimental import pallas as pl
from jax.experimental.pallas import tpu as pltpu
from jax.experimental.pallas import tpu_sc as plsc

_N = 10000
_E = 320000
_F = 128
_H = 512
_G = 64

_NC = 2
_NS = 16
_NW = _NC * _NS
_K = 128
_CPS = 40
_C0 = 120
_C1 = 40
_CH_MAX = max(_C0, _C1)
_E0 = _NS * _C0 * _K
_E1 = _NS * _C1 * _K
_EPAD = _E0 + _E1
_NPAD = 10240
_RPT = _NPAD // _NS

_BM = 1000


@functools.cache
def _get_mp_pass(nt):
    return pl.kernel(
        functools.partial(_mp_body, nt),
        out_type=jax.ShapeDtypeStruct((nt, _NC, _NPAD, _F), jnp.float32),
        mesh=plsc.VectorSubcoreMesh(core_axis_name="c", subcore_axis_name="s",
                                    num_cores=_NC, num_subcores=_NS),
        scratch_types=[
            pltpu.VMEM_SHARED((_NPAD, _F), jnp.float32),
            pltpu.VMEM((_CPS, _K), jnp.int32),
            pltpu.VMEM((_CPS, _K), jnp.int32),
            pltpu.VMEM((_CPS, _K), jnp.float32),
            pltpu.VMEM((2, _K, _F), jnp.float32),
            pltpu.SemaphoreType.DMA,
            pltpu.SemaphoreType.DMA,
            pltpu.SemaphoreType.DMA,
            pltpu.SemaphoreType.DMA,
        ],
    )


def _mp_body(nt, tables, src, dst, w, out, accum, src_t, dst_t, w_t,
             rows_v, sg0, sg1, ss0, ss1):
    cid = lax.axis_index("c")
    sid = lax.axis_index("s")
    wid = cid * _NS + sid
    nstg = jnp.where(cid == 0, _C0 // _CPS, _C1 // _CPS)

    sgs = (sg0, sg1)
    sss = (ss0, ss1)

    @pl.loop(0, nt)
    def run_pass(t):
        @pl.loop(0, _K)
        def _zr(r):
            for v in range(_F // 16):
                rows_v[0, r, pl.ds(v * 16, 16)] = jnp.zeros((16,), jnp.float32)

        for j in range(_RPT // _K):
            pltpu.sync_copy(rows_v.at[0],
                            accum.at[pl.ds(sid * _RPT + j * _K, _K)])
        plsc.subcore_barrier()

        def gather(ci, b):
            return pltpu.async_copy(tables.at[t].at[src_t.at[ci]],
                                    rows_v.at[b], sgs[b])

        def gather_wait(ci, b):
            pltpu.make_async_copy(tables.at[t].at[src_t.at[ci]],
                                  rows_v.at[b], sgs[b]).wait()

        def scatter(ci, b):
            return pltpu.async_copy(rows_v.at[b], accum.at[dst_t.at[ci]],
                                    sss[b], add=True)

        def scatter_wait(ci, b):
            pltpu.make_async_copy(rows_v.at[b], accum.at[dst_t.at[ci]],
                                  sss[b]).wait()

        @pl.loop(0, nstg)
        def _stage(s):
            stg = pl.ds(s * _CPS, _CPS)
            pltpu.sync_copy(src.at[wid, stg], src_t)
            pltpu.sync_copy(dst.at[wid, stg], dst_t)
            pltpu.sync_copy(w.at[wid, stg], w_t)

            gather(0, 0)

            @pl.loop(0, _CPS, step=2)
            def _it(i):
                for b in range(2):
                    ci = i + b
                    gather_wait(ci, b)

                    @plsc.parallel_loop(0, _K // 16)
                    def _grp(g):
                        wvec = w_t[ci, pl.ds(g * 16, 16)]
                        for j in range(16):
                            wk = wvec[j]
                            k = g * 16 + j
                            for v in range(_F // 16):
                                sl = pl.ds(v * 16, 16)
                                rows_v[b, k, sl] = rows_v[b, k, sl] * wk

                    scatter(ci, b)

                    @pl.when(ci < _CPS - 1)
                    def _prefetch():
                        @pl.when(ci >= 1)
                        def _drain():
                            scatter_wait(ci - 1, b ^ 1)
                        gather(ci + 1, b ^ 1)

            scatter_wait(_CPS - 2, 0)
            scatter_wait(_CPS - 1, 1)

        plsc.subcore_barrier()
        pltpu.sync_copy(accum.at[pl.ds(sid * _RPT, _RPT)],
                        out.at[t, cid, pl.ds(sid * _RPT, _RPT)])


def _dense1_body(p_ref, x_ref, wrel_ref, b_ref, wroot_ref, out_ref):
    s = p_ref[0, 0] + p_ref[0, 1]
    xb = x_ref[...]
    for c in range(4):
        wslice = slice(c * _F, (c + 1) * _F)
        acc = jnp.dot(s, wrel_ref[:, wslice], preferred_element_type=jnp.float32)
        acc = acc + jnp.dot(xb, wroot_ref[:, wslice],
                            preferred_element_type=jnp.float32)
        acc = acc + b_ref[0, wslice][None, :]
        out_ref[c] = jnp.maximum(acc, 0.0)


def _dense1(p, x, w_rel, b, w_root):
    return pl.pallas_call(
        _dense1_body,
        grid=(_N // _BM,),
        in_specs=[
            pl.BlockSpec((1, _NC, _BM, _F), lambda m: (0, 0, m, 0)),
            pl.BlockSpec((_BM, _F), lambda m: (m, 0)),
            pl.BlockSpec((_F, _H), lambda m: (0, 0)),
            pl.BlockSpec((1, _H), lambda m: (0, 0)),
            pl.BlockSpec((_F, _H), lambda m: (0, 0)),
        ],
        out_specs=pl.BlockSpec((4, _BM, _F), lambda m: (0, m, 0)),
        out_shape=jax.ShapeDtypeStruct((4, _N, _F), jnp.float32),
    )(p, x, w_rel, b.reshape(1, _H), w_root)


def _dense2_body(q_ref, h_ref, wrel_ref, b_ref, wroot_ref,
                 bt_ref, wl1, bl1, wl2, bl2, wl3, bl3, out_ref,
                 pooled, counts):
    m = pl.program_id(0)
    nblocks = pl.num_programs(0)

    acc = jnp.broadcast_to(b_ref[0][None, :], (_BM, _H))
    for c in range(4):
        ksl = slice(c * _F, (c + 1) * _F)
        aggc = q_ref[c, 0] + q_ref[c, 1]
        acc = acc + jnp.dot(aggc, wrel_ref[ksl, :],
                            preferred_element_type=jnp.float32)
        acc = acc + jnp.dot(h_ref[c], wroot_ref[ksl, :],
                            preferred_element_type=jnp.float32)
    hout = jnp.maximum(acc, 0.0)

    bt = bt_ref[0, 0, :]
    onehot_t = (lax.broadcasted_iota(jnp.int32, (128, _BM), 0)
                == bt[None, :]).astype(jnp.float32)

    @pl.when(m == 0)
    def _init():
        pooled[...] = jnp.zeros_like(pooled)
        counts[...] = jnp.zeros_like(counts)

    pooled[...] += jnp.dot(onehot_t, hout, preferred_element_type=jnp.float32)
    counts[...] += jnp.broadcast_to(
        jnp.sum(onehot_t, axis=1, keepdims=True), (128, 128))

    @pl.when(m == nblocks - 1)
    def _final():
        cnt = counts[:, 0:1]
        mean = pooled[...] / jnp.maximum(cnt, 1.0)
        r = jnp.maximum(jnp.dot(mean, wl1[...],
                                preferred_element_type=jnp.float32)
                        + bl1[0][None, :], 0.0)
        r = jnp.maximum(jnp.dot(r, wl2[...],
                                preferred_element_type=jnp.float32)
                        + bl2[0][None, :], 0.0)
        o = jnp.dot(r, wl3[...], preferred_element_type=jnp.float32) \
            + bl3[0][None, :]
        out_ref[...] = jnp.broadcast_to(o[:_G, :], (_G, 128))


def _dense2(q, h1s, w_rel, b, w_root, bt3, wl1, bl1, wl2, bl2, wl3, bl3):
    full2 = lambda a, b_: pl.BlockSpec((a, b_), lambda m: (0, 0))
    return pl.pallas_call(
        _dense2_body,
        grid=(_N // _BM,),
        in_specs=[
            pl.BlockSpec((4, _NC, _BM, _F), lambda m: (0, 0, m, 0)),
            pl.BlockSpec((4, _BM, _F), lambda m: (0, m, 0)),
            full2(_H, _H),
            full2(1, _H),
            full2(_H, _H),
            pl.BlockSpec((1, 1, _BM), lambda m: (m, 0, 0)),
            full2(_H, 64),
            full2(1, 64),
            full2(64, 16),
            full2(1, 16),
            full2(16, 1),
            full2(1, 1),
        ],
        out_specs=pl.BlockSpec((_G, 128), lambda m: (0, 0)),
        out_shape=jax.ShapeDtypeStruct((_G, 128), jnp.float32),
        scratch_shapes=[
            pltpu.VMEM((128, _H), jnp.float32),
            pltpu.VMEM((128, 128), jnp.float32),
        ],
    )(q, h1s, w_rel, b.reshape(1, _H), w_root, bt3,
      wl1, bl1.reshape(1, 64), wl2, bl2.reshape(1, 16),
      wl3, bl3.reshape(1, 1))


def kernel(x, edge_index, edge_attr, batch, W1_rel, b1, W1_root, W2_rel, b2,
           W2_root, Wl1, bl1, Wl2, bl2, Wl3, bl3):
    def split3(a):
        a = jnp.pad(a, (0, _EPAD - _E))
        c0 = a[:_E0].reshape(_NS, _C0, _K)
        c0 = jnp.pad(c0, ((0, 0), (0, _CH_MAX - _C0), (0, 0)))
        c1 = a[_E0:].reshape(_NS, _C1, _K)
        c1 = jnp.pad(c1, ((0, 0), (0, _CH_MAX - _C1), (0, 0)))
        return jnp.concatenate([c0, c1], axis=0)

    src = split3(edge_index[0])
    dst = split3(edge_index[1])
    w = split3(edge_attr)

    p = _get_mp_pass(1)(x[None], src, dst, w)
    h1s = _dense1(p, x, W1_rel, b1, W1_root)
    q = _get_mp_pass(4)(h1s, src, dst, w)
    bt3 = batch.reshape(_N // _BM, 1, _BM)
    out = _dense2(q, h1s, W2_rel, b2, W2_root, bt3,
                  Wl1, bl1, Wl2, bl2, Wl3, bl3)
    return out[:, :1]
